# Initial kernel scaffold; baseline (speedup 1.0000x reference)
import jax
import jax.numpy as jnp
from jax.experimental import pallas as pl


def kernel(token_ids, embeddings):
    raise NotImplementedError("write your pallas kernel here")


# speedup vs baseline: 6.3329x; 6.3329x over previous
import functools

import jax
import jax.numpy as jnp
from jax import lax
from jax.experimental import pallas as pl
from jax.experimental.pallas import tpu as pltpu
from jax.experimental.pallas import tpu_sc as plsc

NUM_EMB = 100000
D = 128
B_TOK, S = 4096, 200
B = B_TOK * S
NC, NS = 2, 16
NW = NC * NS
BPW = B // NW
C = 128
NCHUNK = BPW // C

_mesh = plsc.VectorSubcoreMesh(core_axis_name="c", subcore_axis_name="s")


@functools.partial(
    pl.kernel,
    mesh=_mesh,
    out_type=jax.ShapeDtypeStruct((B, D), jnp.float32),
    scratch_types=[
        pltpu.VMEM((NCHUNK, C), jnp.int32),
        pltpu.VMEM((C, D), jnp.float32),
        pltpu.SemaphoreType.DMA,
    ],
)
def _emb_lookup(idx_hbm, tab_hbm, out_hbm, idx_v, rows_v, sem):
    wid = lax.axis_index("s") * NC + lax.axis_index("c")
    pltpu.sync_copy(idx_hbm.at[wid], idx_v)

    def body(j, carry):
        pltpu.async_copy(tab_hbm.at[idx_v.at[j]], rows_v, sem).wait()
        pltpu.sync_copy(rows_v, out_hbm.at[pl.ds(wid * BPW + j * C, C)])
        return carry

    lax.fori_loop(0, NCHUNK, body, 0)


def kernel(token_ids, embeddings):
    idx = token_ids.reshape(NW, NCHUNK, C)
    out = _emb_lookup(idx, embeddings)
    return out.reshape(B_TOK, S, D)


# speedup vs baseline: 9.2067x; 1.4538x over previous
import functools

import jax
import jax.numpy as jnp
from jax import lax
from jax.experimental import pallas as pl
from jax.experimental.pallas import tpu as pltpu
from jax.experimental.pallas import tpu_sc as plsc

NUM_EMB = 100000
D = 128
B_TOK, S = 4096, 200
B = B_TOK * S
NC, NS = 2, 16
NW = NC * NS
BPW = B // NW
C = 128
NCHUNK = BPW // C

_mesh = plsc.VectorSubcoreMesh(core_axis_name="c", subcore_axis_name="s")


@functools.partial(
    pl.kernel,
    mesh=_mesh,
    out_type=jax.ShapeDtypeStruct((B, D), jnp.float32),
    scratch_types=[
        pltpu.VMEM((NCHUNK, C), jnp.int32),
        pltpu.VMEM((2, C, D), jnp.float32),
        pltpu.SemaphoreType.DMA,
        pltpu.SemaphoreType.DMA,
        pltpu.SemaphoreType.DMA,
        pltpu.SemaphoreType.DMA,
    ],
)
def _emb_lookup(idx_hbm, tab_hbm, out_hbm, idx_v, rows_v, gs0, gs1, ws0, ws1):
    wid = lax.axis_index("s") * NC + lax.axis_index("c")
    base = wid * BPW
    gsem = (gs0, gs1)
    wsem = (ws0, ws1)
    pltpu.sync_copy(idx_hbm.at[wid], idx_v)

    for b in range(2):
        pltpu.async_copy(tab_hbm.at[idx_v.at[b]], rows_v.at[b], gsem[b])

    def body(jj, carry):
        for b in range(2):
            j = 2 * jj + b
            pltpu.make_async_copy(
                tab_hbm.at[idx_v.at[j]], rows_v.at[b], gsem[b]
            ).wait()
            pltpu.async_copy(
                rows_v.at[b], out_hbm.at[pl.ds(base + j * C, C)], wsem[b]
            )
            pltpu.make_async_copy(
                rows_v.at[b], out_hbm.at[pl.ds(base + j * C, C)], wsem[b]
            ).wait()
            pltpu.async_copy(tab_hbm.at[idx_v.at[j + 2]], rows_v.at[b], gsem[b])
        return carry

    lax.fori_loop(0, (NCHUNK - 2) // 2, body, 0)

    for b in range(2):
        j = NCHUNK - 2 + b
        pltpu.make_async_copy(
            tab_hbm.at[idx_v.at[j]], rows_v.at[b], gsem[b]
        ).wait()
        pltpu.sync_copy(rows_v.at[b], out_hbm.at[pl.ds(base + j * C, C)])


def kernel(token_ids, embeddings):
    idx = token_ids.reshape(NW, NCHUNK, C)
    out = _emb_lookup(idx, embeddings)
    return out.reshape(B_TOK, S, D)
